# Initial kernel scaffold; baseline (speedup 1.0000x reference)
import jax
import jax.numpy as jnp
from jax.experimental import pallas as pl


def kernel(x, edge_index, W1_l, b1, W1_r, W2_l, b2, W2_r):
    raise NotImplementedError("write your pallas kernel here")


# speedup vs baseline: 3.6683x; 3.6683x over previous
import functools

import jax
import jax.numpy as jnp
from jax import lax
from jax.experimental import pallas as pl
from jax.experimental.pallas import tpu as pltpu
from jax.experimental.pallas import tpu_sc as plsc

N = 10000
IN_C = 128
H_C = 256
OUT_C = 128
E = 320000

NC = 2
NS = 16
NW = NC * NS

CHUNK = 128
PER_W = 80
E_PAD = NW * PER_W * CHUNK
ROWS = E_PAD // CHUNK
IDXB = 16
N_PAD = 10112
PER_S = N_PAD // NS


def _make_sc_agg(with_counts):
  mesh = plsc.VectorSubcoreMesh(core_axis_name="c", subcore_axis_name="s")
  out_type = [jax.ShapeDtypeStruct((NC, N_PAD, 128), jnp.float32)]
  scratch = [
      pltpu.VMEM_SHARED((N_PAD, 128), jnp.float32),
      pltpu.VMEM((IDXB, CHUNK), jnp.int32),
      pltpu.VMEM((IDXB, CHUNK), jnp.int32),
      pltpu.VMEM((CHUNK, 128), jnp.float32),
      pltpu.SemaphoreType.DMA,
  ]
  if with_counts:
    out_type.append(jax.ShapeDtypeStruct((NW, N_PAD), jnp.float32))
    scratch.append(pltpu.VMEM((N_PAD,), jnp.float32))

  def body(*refs):
    if with_counts:
      (table, srcr, dstr, z128, agg_out, cnt_out,
       acc, sidx, didx, rows, sem, cnt) = refs
    else:
      (table, srcr, dstr, z128, agg_out,
       acc, sidx, didx, rows, sem) = refs
    c = lax.axis_index("c")
    s = lax.axis_index("s")
    wid = c * NS + s

    pltpu.sync_copy(z128.at[pl.ds(s * PER_S, PER_S)],
                    acc.at[pl.ds(s * PER_S, PER_S)])
    if with_counts:
      def zstep(i, carry):
        cnt[pl.ds(i * 16, 16)] = jnp.zeros((16,), jnp.float32)
        return carry
      lax.fori_loop(0, N_PAD // 16, zstep, 0)
    plsc.subcore_barrier()

    one16 = jnp.ones((16,), jnp.float32)

    def step(j, carry):
      pltpu.async_copy(table.at[sidx.at[j]], rows, sem).wait()
      pltpu.sync_copy(rows, acc.at[didx.at[j]], add=True)
      if with_counts:
        for k in range(CHUNK // 16):
          idx16 = didx[j, pl.ds(k * 16, 16)]
          plsc.addupdate_scatter(cnt, [idx16], one16)
      return carry

    for blk in range(PER_W // IDXB):
      pltpu.sync_copy(srcr.at[wid, pl.ds(blk * IDXB, IDXB)], sidx)
      pltpu.sync_copy(dstr.at[wid, pl.ds(blk * IDXB, IDXB)], didx)
      lax.fori_loop(0, IDXB, step, 0)
    plsc.subcore_barrier()

    pltpu.sync_copy(acc.at[pl.ds(s * PER_S, PER_S)],
                    agg_out.at[c, pl.ds(s * PER_S, PER_S)])
    if with_counts:
      pltpu.sync_copy(cnt, cnt_out.at[wid])

  return pl.kernel(
      body, out_type=out_type, mesh=mesh, scratch_types=scratch,
      compiler_params=pltpu.CompilerParams(needs_layout_passes=False))


_R = 2528


def _tc_layer1(aggp, cntt, xp, w1l, b1, w1r, w2l, b2, w2r):
  def tc_body(aggp_r, cntt_r, x_r, w1l_r, b1_r, w1r_r, w2l_r, b2_r, w2r_r,
              g_r, hr_r):
    agg = aggp_r[0] + aggp_r[1]
    cnt = jnp.sum(cntt_r[...], axis=1, keepdims=True)
    mean = agg * (1.0 / jnp.maximum(cnt, 1.0))
    h = (jnp.dot(mean, w1l_r[...], preferred_element_type=jnp.float32)
         + b1_r[...]
         + jnp.dot(x_r[...], w1r_r[...], preferred_element_type=jnp.float32))
    h = jnp.where(h >= 0, h, 0.01 * h)
    g_r[...] = jnp.dot(h, w2l_r[...], preferred_element_type=jnp.float32)
    hr_r[...] = (jnp.dot(h, w2r_r[...], preferred_element_type=jnp.float32)
                 + b2_r[...])

  grid = (N_PAD // _R,)
  return pl.pallas_call(
      tc_body,
      grid=grid,
      in_specs=[
          pl.BlockSpec((NC, _R, 128), lambda i: (0, i, 0)),
          pl.BlockSpec((_R, NW), lambda i: (i, 0)),
          pl.BlockSpec((_R, 128), lambda i: (i, 0)),
          pl.BlockSpec((IN_C, H_C), lambda i: (0, 0)),
          pl.BlockSpec((1, H_C), lambda i: (0, 0)),
          pl.BlockSpec((IN_C, H_C), lambda i: (0, 0)),
          pl.BlockSpec((H_C, OUT_C), lambda i: (0, 0)),
          pl.BlockSpec((1, OUT_C), lambda i: (0, 0)),
          pl.BlockSpec((H_C, OUT_C), lambda i: (0, 0)),
      ],
      out_specs=[
          pl.BlockSpec((_R, 128), lambda i: (i, 0)),
          pl.BlockSpec((_R, 128), lambda i: (i, 0)),
      ],
      out_shape=[
          jax.ShapeDtypeStruct((N_PAD, OUT_C), jnp.float32),
          jax.ShapeDtypeStruct((N_PAD, OUT_C), jnp.float32),
      ],
  )(aggp, cntt, xp, w1l, b1, w1r, w2l, b2, w2r)


def _tc_layer2(agg2p, cntt, hr):
  def tc_body(aggp_r, cntt_r, hr_r, out_r):
    agg = aggp_r[0] + aggp_r[1]
    cnt = jnp.sum(cntt_r[...], axis=1, keepdims=True)
    out_r[...] = agg * (1.0 / jnp.maximum(cnt, 1.0)) + hr_r[...]

  grid = (N_PAD // _R,)
  return pl.pallas_call(
      tc_body,
      grid=grid,
      in_specs=[
          pl.BlockSpec((NC, _R, 128), lambda i: (0, i, 0)),
          pl.BlockSpec((_R, NW), lambda i: (i, 0)),
          pl.BlockSpec((_R, 128), lambda i: (i, 0)),
      ],
      out_specs=pl.BlockSpec((_R, 128), lambda i: (i, 0)),
      out_shape=jax.ShapeDtypeStruct((N_PAD, OUT_C), jnp.float32),
  )(agg2p, cntt, hr)


_sc_agg_counts = _make_sc_agg(True)
_sc_agg_plain = _make_sc_agg(False)


def kernel(x, edge_index, W1_l, b1, W1_r, W2_l, b2, W2_r):
  src = edge_index[0].astype(jnp.int32)
  dst = edge_index[1].astype(jnp.int32)
  pad_e = E_PAD - E
  srcr = jnp.concatenate([src, jnp.zeros((pad_e,), jnp.int32)]).reshape(
      NW, PER_W, CHUNK)
  dstr = jnp.concatenate([dst, jnp.full((pad_e,), N, jnp.int32)]).reshape(
      NW, PER_W, CHUNK)
  xp = jnp.zeros((N_PAD, IN_C), jnp.float32).at[:N].set(x)
  z128 = jnp.zeros((N_PAD, 128), jnp.float32)

  aggp, cntp = _sc_agg_counts(xp, srcr, dstr, z128)
  cntt = cntp.T
  g, hr = _tc_layer1(aggp, cntt, xp,
                     W1_l, b1.reshape(1, H_C), W1_r,
                     W2_l, b2.reshape(1, OUT_C), W2_r)
  (agg2p,) = _sc_agg_plain(g, srcr, dstr, z128)
  out = _tc_layer2(agg2p, cntt, hr)
  return out[:N]


# speedup vs baseline: 4.0214x; 1.0963x over previous
import functools

import jax
import jax.numpy as jnp
from jax import lax
from jax.experimental import pallas as pl
from jax.experimental.pallas import tpu as pltpu
from jax.experimental.pallas import tpu_sc as plsc

N = 10000
IN_C = 128
H_C = 256
OUT_C = 128
E = 320000

NC = 2
NS = 16
NW = NC * NS

CHUNK = 128
PER_W = 80
E_PAD = NW * PER_W * CHUNK
ROWS = E_PAD // CHUNK
IDXB = 8
N_PAD = 10112
PER_S = N_PAD // NS


def _make_sc_agg(with_counts):
  mesh = plsc.VectorSubcoreMesh(core_axis_name="c", subcore_axis_name="s")
  out_type = [jax.ShapeDtypeStruct((NC, N_PAD, 128), jnp.float32)]
  scratch = [
      pltpu.VMEM_SHARED((N_PAD, 128), jnp.float32),
      pltpu.VMEM((2, IDXB, CHUNK), jnp.int32),
      pltpu.VMEM((2, IDXB, CHUNK), jnp.int32),
      pltpu.VMEM((2, CHUNK, 128), jnp.float32),
      pltpu.SemaphoreType.DMA,
      pltpu.SemaphoreType.DMA,
  ]
  if with_counts:
    out_type.append(jax.ShapeDtypeStruct((NW, N_PAD), jnp.float32))
    scratch.append(pltpu.VMEM((N_PAD,), jnp.float32))

  NBLK = PER_W // IDXB

  def body(*refs):
    if with_counts:
      (table, srcr, dstr, z128, agg_out, cnt_out,
       acc, sidx, didx, rows, sem_g, sem_i, cnt) = refs
    else:
      (table, srcr, dstr, z128, agg_out,
       acc, sidx, didx, rows, sem_g, sem_i) = refs
    c = lax.axis_index("c")
    s = lax.axis_index("s")
    wid = c * NS + s

    pltpu.sync_copy(z128.at[pl.ds(s * PER_S, PER_S)],
                    acc.at[pl.ds(s * PER_S, PER_S)])
    if with_counts:
      def zstep(i, carry):
        cnt[pl.ds(i * 16, 16)] = jnp.zeros((16,), jnp.float32)
        return carry
      lax.fori_loop(0, N_PAD // 16, zstep, 0)

    pltpu.sync_copy(srcr.at[wid, pl.ds(0, IDXB)], sidx.at[0])
    pltpu.sync_copy(dstr.at[wid, pl.ds(0, IDXB)], didx.at[0])
    pltpu.async_copy(table.at[sidx.at[0, 0]], rows.at[0], sem_g)
    plsc.subcore_barrier()

    one16 = jnp.ones((16,), jnp.float32)

    def step(j, carry):
      p = j % 2
      b = j // IDXB
      bp = b % 2
      r = j % IDXB
      jn = j + 1

      @pl.when(jnp.logical_and(r == 0, b + 1 < NBLK))
      def _stage_next():
        pltpu.async_copy(srcr.at[wid, pl.ds((b + 1) * IDXB, IDXB)],
                         sidx.at[1 - bp], sem_i)
        pltpu.async_copy(dstr.at[wid, pl.ds((b + 1) * IDXB, IDXB)],
                         didx.at[1 - bp], sem_i)

      pltpu.make_async_copy(table.at[sidx.at[bp, r]], rows.at[p], sem_g).wait()

      @pl.when(jnp.logical_and(r == IDXB - 1, b + 1 < NBLK))
      def _wait_next_idx():
        pltpu.make_async_copy(srcr.at[wid, pl.ds((b + 1) * IDXB, IDXB)],
                              sidx.at[1 - bp], sem_i).wait()
        pltpu.make_async_copy(dstr.at[wid, pl.ds((b + 1) * IDXB, IDXB)],
                              didx.at[1 - bp], sem_i).wait()

      @pl.when(jn < PER_W)
      def _fire_next():
        pltpu.async_copy(table.at[sidx.at[(jn // IDXB) % 2, jn % IDXB]],
                         rows.at[1 - p], sem_g)

      pltpu.sync_copy(rows.at[p], acc.at[didx.at[bp, r]], add=True)
      if with_counts:
        for k in range(CHUNK // 16):
          idx16 = didx[bp, r, pl.ds(k * 16, 16)]
          plsc.addupdate_scatter(cnt, [idx16], one16)
      return carry

    lax.fori_loop(0, PER_W, step, 0)
    plsc.subcore_barrier()

    pltpu.sync_copy(acc.at[pl.ds(s * PER_S, PER_S)],
                    agg_out.at[c, pl.ds(s * PER_S, PER_S)])
    if with_counts:
      pltpu.sync_copy(cnt, cnt_out.at[wid])

  return pl.kernel(
      body, out_type=out_type, mesh=mesh, scratch_types=scratch,
      compiler_params=pltpu.CompilerParams(needs_layout_passes=False))


_R = 2528


def _tc_layer1(aggp, cntt, xp, w1l, b1, w1r, w2l, b2, w2r):
  def tc_body(aggp_r, cntt_r, x_r, w1l_r, b1_r, w1r_r, w2l_r, b2_r, w2r_r,
              g_r, hr_r):
    agg = aggp_r[0] + aggp_r[1]
    cnt = jnp.sum(cntt_r[...], axis=1, keepdims=True)
    mean = agg * (1.0 / jnp.maximum(cnt, 1.0))
    h = (jnp.dot(mean, w1l_r[...], preferred_element_type=jnp.float32)
         + b1_r[...]
         + jnp.dot(x_r[...], w1r_r[...], preferred_element_type=jnp.float32))
    h = jnp.where(h >= 0, h, 0.01 * h)
    g_r[...] = jnp.dot(h, w2l_r[...], preferred_element_type=jnp.float32)
    hr_r[...] = (jnp.dot(h, w2r_r[...], preferred_element_type=jnp.float32)
                 + b2_r[...])

  grid = (N_PAD // _R,)
  return pl.pallas_call(
      tc_body,
      grid=grid,
      in_specs=[
          pl.BlockSpec((NC, _R, 128), lambda i: (0, i, 0)),
          pl.BlockSpec((_R, NW), lambda i: (i, 0)),
          pl.BlockSpec((_R, 128), lambda i: (i, 0)),
          pl.BlockSpec((IN_C, H_C), lambda i: (0, 0)),
          pl.BlockSpec((1, H_C), lambda i: (0, 0)),
          pl.BlockSpec((IN_C, H_C), lambda i: (0, 0)),
          pl.BlockSpec((H_C, OUT_C), lambda i: (0, 0)),
          pl.BlockSpec((1, OUT_C), lambda i: (0, 0)),
          pl.BlockSpec((H_C, OUT_C), lambda i: (0, 0)),
      ],
      out_specs=[
          pl.BlockSpec((_R, 128), lambda i: (i, 0)),
          pl.BlockSpec((_R, 128), lambda i: (i, 0)),
      ],
      out_shape=[
          jax.ShapeDtypeStruct((N_PAD, OUT_C), jnp.float32),
          jax.ShapeDtypeStruct((N_PAD, OUT_C), jnp.float32),
      ],
  )(aggp, cntt, xp, w1l, b1, w1r, w2l, b2, w2r)


def _tc_layer2(agg2p, cntt, hr):
  def tc_body(aggp_r, cntt_r, hr_r, out_r):
    agg = aggp_r[0] + aggp_r[1]
    cnt = jnp.sum(cntt_r[...], axis=1, keepdims=True)
    out_r[...] = agg * (1.0 / jnp.maximum(cnt, 1.0)) + hr_r[...]

  grid = (N_PAD // _R,)
  return pl.pallas_call(
      tc_body,
      grid=grid,
      in_specs=[
          pl.BlockSpec((NC, _R, 128), lambda i: (0, i, 0)),
          pl.BlockSpec((_R, NW), lambda i: (i, 0)),
          pl.BlockSpec((_R, 128), lambda i: (i, 0)),
      ],
      out_specs=pl.BlockSpec((_R, 128), lambda i: (i, 0)),
      out_shape=jax.ShapeDtypeStruct((N_PAD, OUT_C), jnp.float32),
  )(agg2p, cntt, hr)


_sc_agg_counts = _make_sc_agg(True)
_sc_agg_plain = _make_sc_agg(False)


def kernel(x, edge_index, W1_l, b1, W1_r, W2_l, b2, W2_r):
  src = edge_index[0].astype(jnp.int32)
  dst = edge_index[1].astype(jnp.int32)
  pad_e = E_PAD - E
  srcr = jnp.concatenate([src, jnp.zeros((pad_e,), jnp.int32)]).reshape(
      NW, PER_W, CHUNK)
  dstr = jnp.concatenate([dst, jnp.full((pad_e,), N, jnp.int32)]).reshape(
      NW, PER_W, CHUNK)
  xp = jnp.zeros((N_PAD, IN_C), jnp.float32).at[:N].set(x)
  z128 = jnp.zeros((N_PAD, 128), jnp.float32)

  aggp, cntp = _sc_agg_counts(xp, srcr, dstr, z128)
  cntt = cntp.T
  g, hr = _tc_layer1(aggp, cntt, xp,
                     W1_l, b1.reshape(1, H_C), W1_r,
                     W2_l, b2.reshape(1, OUT_C), W2_r)
  (agg2p,) = _sc_agg_plain(g, srcr, dstr, z128)
  out = _tc_layer2(agg2p, cntt, hr)
  return out[:N]


# speedup vs baseline: 4.1531x; 1.0327x over previous
import functools

import jax
import jax.numpy as jnp
from jax import lax
from jax.experimental import pallas as pl
from jax.experimental.pallas import tpu as pltpu
from jax.experimental.pallas import tpu_sc as plsc

N = 10000
IN_C = 128
H_C = 256
OUT_C = 128
E = 320000

NC = 2
NS = 16
NW = NC * NS

CHUNK = 128
PER_W = 80
E_PAD = NW * PER_W * CHUNK
ROWS = E_PAD // CHUNK
IDXB = 8
N_PAD = 10112
PER_S = N_PAD // NS


def _make_sc_agg(with_counts):
  mesh = plsc.VectorSubcoreMesh(core_axis_name="c", subcore_axis_name="s")
  out_type = [jax.ShapeDtypeStruct((NC, N_PAD, 128), jnp.float32)]
  scratch = [
      pltpu.VMEM_SHARED((N_PAD, 128), jnp.float32),
      pltpu.VMEM((2, IDXB, CHUNK), jnp.int32),
      pltpu.VMEM((2, IDXB, CHUNK), jnp.int32),
      pltpu.VMEM((2, CHUNK, 128), jnp.float32),
      pltpu.SemaphoreType.DMA,
      pltpu.SemaphoreType.DMA,
  ]
  if with_counts:
    out_type.append(jax.ShapeDtypeStruct((NW, N_PAD), jnp.float32))
    scratch.append(pltpu.VMEM((N_PAD,), jnp.float32))

  NBLK = PER_W // IDXB

  def body(*refs):
    if with_counts:
      (table, srcr, dstr, z128, agg_out, cnt_out,
       acc, sidx, didx, rows, sem_g, sem_i, cnt) = refs
    else:
      (table, srcr, dstr, z128, agg_out,
       acc, sidx, didx, rows, sem_g, sem_i) = refs
    c = lax.axis_index("c")
    s = lax.axis_index("s")
    wid = c * NS + s

    pltpu.sync_copy(z128.at[pl.ds(s * PER_S, PER_S)],
                    acc.at[pl.ds(s * PER_S, PER_S)])
    if with_counts:
      def zstep(i, carry):
        cnt[pl.ds(i * 16, 16)] = jnp.zeros((16,), jnp.float32)
        return carry
      lax.fori_loop(0, N_PAD // 16, zstep, 0)

    pltpu.sync_copy(srcr.at[wid, pl.ds(0, IDXB)], sidx.at[0])
    pltpu.sync_copy(dstr.at[wid, pl.ds(0, IDXB)], didx.at[0])
    pltpu.async_copy(table.at[sidx.at[0, 0, pl.ds(0, 64)]],
                     rows.at[0, pl.ds(0, 64)], sem_g)
    pltpu.async_copy(table.at[sidx.at[0, 0, pl.ds(64, 64)]],
                     rows.at[0, pl.ds(64, 64)], sem_g)
    plsc.subcore_barrier()

    one16 = jnp.ones((16,), jnp.float32)

    def step(j, carry):
      p = j % 2
      b = j // IDXB
      bp = b % 2
      r = j % IDXB
      jn = j + 1

      @pl.when(jnp.logical_and(r == 0, b + 1 < NBLK))
      def _stage_next():
        pltpu.async_copy(srcr.at[wid, pl.ds((b + 1) * IDXB, IDXB)],
                         sidx.at[1 - bp], sem_i)
        pltpu.async_copy(dstr.at[wid, pl.ds((b + 1) * IDXB, IDXB)],
                         didx.at[1 - bp], sem_i)

      @pl.when(jnp.logical_and(r == IDXB - 1, b + 1 < NBLK))
      def _wait_next_idx():
        pltpu.make_async_copy(srcr.at[wid, pl.ds((b + 1) * IDXB, IDXB)],
                              sidx.at[1 - bp], sem_i).wait()
        pltpu.make_async_copy(dstr.at[wid, pl.ds((b + 1) * IDXB, IDXB)],
                              didx.at[1 - bp], sem_i).wait()

      @pl.when(jn < PER_W)
      def _fire_next():
        bn = (jn // IDXB) % 2
        rn = jn % IDXB
        pltpu.async_copy(table.at[sidx.at[bn, rn, pl.ds(0, 64)]],
                         rows.at[1 - p, pl.ds(0, 64)], sem_g)
        pltpu.async_copy(table.at[sidx.at[bn, rn, pl.ds(64, 64)]],
                         rows.at[1 - p, pl.ds(64, 64)], sem_g)

      pltpu.make_async_copy(table.at[sidx.at[bp, r, pl.ds(0, 64)]],
                            rows.at[p, pl.ds(0, 64)], sem_g).wait()
      pltpu.make_async_copy(table.at[sidx.at[bp, r, pl.ds(64, 64)]],
                            rows.at[p, pl.ds(64, 64)], sem_g).wait()

      pltpu.sync_copy(rows.at[p], acc.at[didx.at[bp, r]], add=True)
      if with_counts:
        for k in range(CHUNK // 16):
          idx16 = didx[bp, r, pl.ds(k * 16, 16)]
          plsc.addupdate_scatter(cnt, [idx16], one16)
      return carry

    lax.fori_loop(0, PER_W, step, 0)
    plsc.subcore_barrier()

    pltpu.sync_copy(acc.at[pl.ds(s * PER_S, PER_S)],
                    agg_out.at[c, pl.ds(s * PER_S, PER_S)])
    if with_counts:
      pltpu.sync_copy(cnt, cnt_out.at[wid])

  return pl.kernel(
      body, out_type=out_type, mesh=mesh, scratch_types=scratch,
      compiler_params=pltpu.CompilerParams(needs_layout_passes=False))


_R = 2528


def _tc_layer1(aggp, cntt, xp, w1l, b1, w1r, w2l, b2, w2r):
  def tc_body(aggp_r, cntt_r, x_r, w1l_r, b1_r, w1r_r, w2l_r, b2_r, w2r_r,
              g_r, hr_r):
    agg = aggp_r[0] + aggp_r[1]
    cnt = jnp.sum(cntt_r[...], axis=1, keepdims=True)
    mean = agg * (1.0 / jnp.maximum(cnt, 1.0))
    h = (jnp.dot(mean, w1l_r[...], preferred_element_type=jnp.float32)
         + b1_r[...]
         + jnp.dot(x_r[...], w1r_r[...], preferred_element_type=jnp.float32))
    h = jnp.where(h >= 0, h, 0.01 * h)
    g_r[...] = jnp.dot(h, w2l_r[...], preferred_element_type=jnp.float32)
    hr_r[...] = (jnp.dot(h, w2r_r[...], preferred_element_type=jnp.float32)
                 + b2_r[...])

  grid = (N_PAD // _R,)
  return pl.pallas_call(
      tc_body,
      grid=grid,
      in_specs=[
          pl.BlockSpec((NC, _R, 128), lambda i: (0, i, 0)),
          pl.BlockSpec((_R, NW), lambda i: (i, 0)),
          pl.BlockSpec((_R, 128), lambda i: (i, 0)),
          pl.BlockSpec((IN_C, H_C), lambda i: (0, 0)),
          pl.BlockSpec((1, H_C), lambda i: (0, 0)),
          pl.BlockSpec((IN_C, H_C), lambda i: (0, 0)),
          pl.BlockSpec((H_C, OUT_C), lambda i: (0, 0)),
          pl.BlockSpec((1, OUT_C), lambda i: (0, 0)),
          pl.BlockSpec((H_C, OUT_C), lambda i: (0, 0)),
      ],
      out_specs=[
          pl.BlockSpec((_R, 128), lambda i: (i, 0)),
          pl.BlockSpec((_R, 128), lambda i: (i, 0)),
      ],
      out_shape=[
          jax.ShapeDtypeStruct((N_PAD, OUT_C), jnp.float32),
          jax.ShapeDtypeStruct((N_PAD, OUT_C), jnp.float32),
      ],
  )(aggp, cntt, xp, w1l, b1, w1r, w2l, b2, w2r)


def _tc_layer2(agg2p, cntt, hr):
  def tc_body(aggp_r, cntt_r, hr_r, out_r):
    agg = aggp_r[0] + aggp_r[1]
    cnt = jnp.sum(cntt_r[...], axis=1, keepdims=True)
    out_r[...] = agg * (1.0 / jnp.maximum(cnt, 1.0)) + hr_r[...]

  grid = (N_PAD // _R,)
  return pl.pallas_call(
      tc_body,
      grid=grid,
      in_specs=[
          pl.BlockSpec((NC, _R, 128), lambda i: (0, i, 0)),
          pl.BlockSpec((_R, NW), lambda i: (i, 0)),
          pl.BlockSpec((_R, 128), lambda i: (i, 0)),
      ],
      out_specs=pl.BlockSpec((_R, 128), lambda i: (i, 0)),
      out_shape=jax.ShapeDtypeStruct((N_PAD, OUT_C), jnp.float32),
  )(agg2p, cntt, hr)


_sc_agg_counts = _make_sc_agg(True)
_sc_agg_plain = _make_sc_agg(False)


def kernel(x, edge_index, W1_l, b1, W1_r, W2_l, b2, W2_r):
  src = edge_index[0].astype(jnp.int32)
  dst = edge_index[1].astype(jnp.int32)
  pad_e = E_PAD - E
  srcr = jnp.concatenate([src, jnp.zeros((pad_e,), jnp.int32)]).reshape(
      NW, PER_W, CHUNK)
  dstr = jnp.concatenate([dst, jnp.full((pad_e,), N, jnp.int32)]).reshape(
      NW, PER_W, CHUNK)
  xp = jnp.zeros((N_PAD, IN_C), jnp.float32).at[:N].set(x)
  z128 = jnp.zeros((N_PAD, 128), jnp.float32)

  aggp, cntp = _sc_agg_counts(xp, srcr, dstr, z128)
  cntt = cntp.T
  g, hr = _tc_layer1(aggp, cntt, xp,
                     W1_l, b1.reshape(1, H_C), W1_r,
                     W2_l, b2.reshape(1, OUT_C), W2_r)
  (agg2p,) = _sc_agg_plain(g, srcr, dstr, z128)
  out = _tc_layer2(agg2p, cntt, hr)
  return out[:N]
